# baseline (speedup 1.0000x reference)
import functools

import jax
from jax import lax
import jax.numpy as jnp
import numpy as np
from jax.experimental import pallas as pl
from jax.experimental.pallas import tpu as pltpu
from jax.experimental.pallas import tpu_sc as plsc

B, U, GH, GW, E, DX, H = 4, 8192, 32, 32, 128, 2, 8
S = GH * GW
DH = E // H
BU = 2048
NU = U // BU
INV_SQRT_DH = 1.0 / np.sqrt(DH)

_SC_INFO = plsc.get_sparse_core_info()
NC, NS, NL = _SC_INFO.num_cores, _SC_INFO.num_subcores, _SC_INFO.num_lanes
NW = NC * NS
TPW = (B * U) // NW
CHUNK = 128
NCH = TPW // CHUNK


def _head_mask():
    r = jax.lax.broadcasted_iota(jnp.int32, (E, E), 0) // DH
    c = jax.lax.broadcasted_iota(jnp.int32, (E, E), 1) // DH
    return (r == c).astype(jnp.float32)


def _prep_kernel(lat_ref, on_ref, wq_ref, wk_ref, wv_ref,
                 qm_ref, eon_ref, von_ref):
    b = pl.program_id(0)
    mhead = _head_mask()
    qm = jnp.dot(lat_ref[...], wq_ref[...], preferred_element_type=jnp.float32)

    @pl.when(b == 0)
    def _():
        qm_ref[...] = qm

    on = on_ref[0]
    kon = jnp.dot(on, wk_ref[...], preferred_element_type=jnp.float32)
    von_ref[0] = jnp.dot(on, wv_ref[...], preferred_element_type=jnp.float32)
    son = jnp.dot(qm * kon, mhead,
                  preferred_element_type=jnp.float32) * INV_SQRT_DH
    eon_ref[0] = jnp.exp(son)


_sc_mesh = plsc.VectorSubcoreMesh(core_axis_name="c", subcore_axis_name="s")


@functools.partial(
    pl.kernel, mesh=_sc_mesh,
    out_type=[jax.ShapeDtypeStruct((B * U, E), jnp.float32),
              jax.ShapeDtypeStruct((B * U,), jnp.int32)],
    scratch_types=[
        pltpu.VMEM((CHUNK,), jnp.float32),
        pltpu.VMEM((CHUNK,), jnp.float32),
        pltpu.VMEM((CHUNK,), jnp.int32),
        pltpu.VMEM((CHUNK, E), jnp.float32),
        pltpu.SemaphoreType.DMA,
    ],
)
def _sc_gather(x_hbm, y_hbm, qm_hbm, qg_hbm, idx_hbm,
               x_v, y_v, idx_v, rows_v, sem):
    wid = lax.axis_index("s") * NC + lax.axis_index("c")
    base = wid * TPW
    for ci in range(NCH):
        off = base + ci * CHUNK
        pltpu.sync_copy(x_hbm.at[pl.ds(off, CHUNK)], x_v)
        pltpu.sync_copy(y_hbm.at[pl.ds(off, CHUNK)], y_v)
        for i in range(CHUNK // NL):
            sl = pl.ds(i * NL, NL)
            tx = jnp.minimum(jnp.maximum(x_v[sl] * (GH - 1) + 0.5, 0.0),
                             GH - 1 + 0.49)
            ty = jnp.minimum(jnp.maximum(y_v[sl] * (GW - 1) + 0.5, 0.0),
                             GW - 1 + 0.49)
            gi = tx.astype(jnp.int32)
            gj = ty.astype(jnp.int32)
            idx_v[sl] = gi * GW + gj
        pltpu.async_copy(qm_hbm.at[idx_v], rows_v, sem).wait()
        pltpu.sync_copy(rows_v, qg_hbm.at[pl.ds(off, CHUNK)])
        pltpu.sync_copy(idx_v, idx_hbm.at[pl.ds(off, CHUNK)])


def _main_kernel(qg_ref, idx_ref, z_ref, eon_ref, von_ref, wk_ref, wv_ref,
                 wo_ref, out_ref, acc_ref, wkv_ref):
    u = pl.program_id(1)
    mhead = _head_mask()

    @pl.when(u == 0)
    def _init():
        wkv_ref[...] = jnp.concatenate([wk_ref[...], wv_ref[...]], axis=1)
        acc_ref[...] = jnp.zeros_like(acc_ref)

    z = z_ref[0]
    qg = qg_ref[0]
    idx_r = idx_ref[0, 0]
    onehot_t = (idx_r == jax.lax.broadcasted_iota(jnp.int32, (S, BU), 0)
                ).astype(jnp.float32)

    kv = jnp.dot(z, wkv_ref[...], preferred_element_type=jnp.float32)
    k, v = kv[:, :E], kv[:, E:]
    scores = jnp.dot(qg * k, mhead,
                     preferred_element_type=jnp.float32) * INV_SQRT_DH
    w = jnp.exp(scores)

    payload = jnp.concatenate([v * w, w], axis=1)
    acc_ref[...] += jnp.dot(onehot_t, payload,
                            preferred_element_type=jnp.float32)

    @pl.when(u == NU - 1)
    def _finalize():
        eon = eon_ref[0]
        num, den = acc_ref[:, :E], acc_ref[:, E:]
        outm = (num + eon * von_ref[0]) / (den + eon)
        out_ref[0] = jnp.dot(outm, wo_ref[...],
                             preferred_element_type=jnp.float32)


def kernel(xc_off_grid, xc_on_grid, zc_off_grid, zc_on_grid, ignore_on_grid,
           latents, fake_embedding, Wq, Wk, Wv, Wo):
    Bv = xc_on_grid.shape[0]
    grid_shape = xc_on_grid.shape[1:-1]
    zc_on = zc_on_grid.reshape(Bv, S, E)
    on_tok = jnp.where(jnp.asarray(ignore_on_grid),
                       jnp.broadcast_to(fake_embedding, (Bv, S, E)), zc_on)

    qm, eon, von = pl.pallas_call(
        _prep_kernel,
        grid=(Bv,),
        in_specs=[
            pl.BlockSpec((S, E), lambda b: (0, 0)),
            pl.BlockSpec((1, S, E), lambda b: (b, 0, 0)),
            pl.BlockSpec((E, E), lambda b: (0, 0)),
            pl.BlockSpec((E, E), lambda b: (0, 0)),
            pl.BlockSpec((E, E), lambda b: (0, 0)),
        ],
        out_specs=[
            pl.BlockSpec((S, E), lambda b: (0, 0)),
            pl.BlockSpec((1, S, E), lambda b: (b, 0, 0)),
            pl.BlockSpec((1, S, E), lambda b: (b, 0, 0)),
        ],
        out_shape=[
            jax.ShapeDtypeStruct((S, E), jnp.float32),
            jax.ShapeDtypeStruct((Bv, S, E), jnp.float32),
            jax.ShapeDtypeStruct((Bv, S, E), jnp.float32),
        ],
    )(latents, on_tok, Wq, Wk, Wv)

    x_flat = xc_off_grid[..., 0].reshape(Bv * U)
    y_flat = xc_off_grid[..., 1].reshape(Bv * U)
    qg_flat, idx_flat = _sc_gather(x_flat, y_flat, qm)

    qg = qg_flat.reshape(Bv, U, E)
    idx4 = idx_flat.reshape(Bv, NU, 1, BU)

    out = pl.pallas_call(
        _main_kernel,
        grid=(Bv, NU),
        in_specs=[
            pl.BlockSpec((1, BU, E), lambda b, u: (b, u, 0)),
            pl.BlockSpec((1, 1, 1, BU), lambda b, u: (b, u, 0, 0)),
            pl.BlockSpec((1, BU, E), lambda b, u: (b, u, 0)),
            pl.BlockSpec((1, S, E), lambda b, u: (b, 0, 0)),
            pl.BlockSpec((1, S, E), lambda b, u: (b, 0, 0)),
            pl.BlockSpec((E, E), lambda b, u: (0, 0)),
            pl.BlockSpec((E, E), lambda b, u: (0, 0)),
            pl.BlockSpec((E, E), lambda b, u: (0, 0)),
        ],
        out_specs=pl.BlockSpec((1, S, E), lambda b, u: (b, 0, 0)),
        out_shape=jax.ShapeDtypeStruct((Bv, S, E), jnp.float32),
        scratch_shapes=[
            pltpu.VMEM((S, 2 * E), jnp.float32),
            pltpu.VMEM((E, 2 * E), jnp.float32),
        ],
        compiler_params=pltpu.CompilerParams(
            dimension_semantics=("parallel", "arbitrary")),
    )(qg, idx4, zc_off_grid, eon, von, Wk, Wv, Wo)

    return out.reshape((Bv,) + tuple(grid_shape) + (E,))


# speedup vs baseline: 1.0715x; 1.0715x over previous
import functools

import jax
from jax import lax
import jax.numpy as jnp
import numpy as np
from jax.experimental import pallas as pl
from jax.experimental.pallas import tpu as pltpu
from jax.experimental.pallas import tpu_sc as plsc

B, U, GH, GW, E, DX, H = 4, 8192, 32, 32, 128, 2, 8
S = GH * GW
DH = E // H
BU = 2048
NU = U // BU
INV_SQRT_DH = 1.0 / np.sqrt(DH)

_SC_INFO = plsc.get_sparse_core_info()
NC, NS, NL = _SC_INFO.num_cores, _SC_INFO.num_subcores, _SC_INFO.num_lanes
NW = NC * NS
TPW = (B * U) // NW
CHUNK = 128
NCH = TPW // CHUNK


def _head_mask():
    r = jax.lax.broadcasted_iota(jnp.int32, (E, E), 0) // DH
    c = jax.lax.broadcasted_iota(jnp.int32, (E, E), 1) // DH
    return (r == c).astype(jnp.float32)


def _prep_kernel(lat_ref, on_ref, wq_ref, wk_ref, wv_ref,
                 qm_ref, eon_ref, von_ref):
    b = pl.program_id(0)
    mhead = _head_mask()
    qm = jnp.dot(lat_ref[...], wq_ref[...], preferred_element_type=jnp.float32)

    @pl.when(b == 0)
    def _():
        qm_ref[...] = qm

    on = on_ref[0]
    kon = jnp.dot(on, wk_ref[...], preferred_element_type=jnp.float32)
    von_ref[0] = jnp.dot(on, wv_ref[...], preferred_element_type=jnp.float32)
    son = jnp.dot(qm * kon, mhead,
                  preferred_element_type=jnp.float32) * INV_SQRT_DH
    eon_ref[0] = jnp.exp(son)


_sc_mesh = plsc.VectorSubcoreMesh(core_axis_name="c", subcore_axis_name="s")


@functools.partial(
    pl.kernel, mesh=_sc_mesh,
    out_type=[jax.ShapeDtypeStruct((B * U, E), jnp.float32),
              jax.ShapeDtypeStruct((B * U,), jnp.int32)],
    scratch_types=[
        pltpu.VMEM((TPW,), jnp.float32),
        pltpu.VMEM((TPW,), jnp.float32),
        pltpu.VMEM((TPW,), jnp.int32),
        pltpu.VMEM((CHUNK, E), jnp.float32),
        pltpu.VMEM((CHUNK, E), jnp.float32),
        pltpu.SemaphoreType.DMA,
        pltpu.SemaphoreType.DMA,
    ],
)
def _sc_gather(x_hbm, y_hbm, qm_hbm, qg_hbm, idx_hbm,
               x_v, y_v, idx_v, rows0, rows1, sem0, sem1):
    wid = lax.axis_index("s") * NC + lax.axis_index("c")
    base = wid * TPW
    pltpu.sync_copy(x_hbm.at[pl.ds(base, TPW)], x_v)
    pltpu.sync_copy(y_hbm.at[pl.ds(base, TPW)], y_v)
    for i in range(TPW // NL):
        sl = pl.ds(i * NL, NL)
        tx = jnp.minimum(jnp.maximum(x_v[sl] * (GH - 1) + 0.5, 0.0),
                         GH - 1 + 0.49)
        ty = jnp.minimum(jnp.maximum(y_v[sl] * (GW - 1) + 0.5, 0.0),
                         GW - 1 + 0.49)
        gi = tx.astype(jnp.int32)
        gj = ty.astype(jnp.int32)
        idx_v[sl] = gi * GW + gj
    pltpu.sync_copy(idx_v, idx_hbm.at[pl.ds(base, TPW)])
    rows = (rows0, rows1)
    sems = (sem0, sem1)
    handles = [None, None]
    for ci in range(NCH):
        bsel = ci % 2
        handles[bsel] = pltpu.async_copy(
            qm_hbm.at[idx_v.at[pl.ds(ci * CHUNK, CHUNK)]], rows[bsel],
            sems[bsel])
        if ci >= 1:
            pb = (ci - 1) % 2
            handles[pb].wait()
            pltpu.sync_copy(rows[pb],
                            qg_hbm.at[pl.ds(base + (ci - 1) * CHUNK, CHUNK)])
    last = (NCH - 1) % 2
    handles[last].wait()
    pltpu.sync_copy(rows[last],
                    qg_hbm.at[pl.ds(base + (NCH - 1) * CHUNK, CHUNK)])


def _main_kernel(qg_ref, idx_ref, z_ref, eon_ref, von_ref, wk_ref, wv_ref,
                 wo_ref, out_ref, acc_ref, wkv_ref):
    u = pl.program_id(1)
    mhead = _head_mask()

    @pl.when(u == 0)
    def _init():
        wkv_ref[...] = jnp.concatenate([wk_ref[...], wv_ref[...]], axis=1)
        acc_ref[...] = jnp.zeros_like(acc_ref)

    z = z_ref[0]
    qg = qg_ref[0]
    idx_r = idx_ref[0, 0]
    onehot_t = (idx_r == jax.lax.broadcasted_iota(jnp.int32, (S, BU), 0)
                ).astype(jnp.float32)

    kv = jnp.dot(z, wkv_ref[...], preferred_element_type=jnp.float32)
    k, v = kv[:, :E], kv[:, E:]
    scores = jnp.dot(qg * k, mhead,
                     preferred_element_type=jnp.float32) * INV_SQRT_DH
    w = jnp.exp(scores)

    payload = jnp.concatenate([v * w, w], axis=1)
    acc_ref[...] += jnp.dot(onehot_t, payload,
                            preferred_element_type=jnp.float32)

    @pl.when(u == NU - 1)
    def _finalize():
        eon = eon_ref[0]
        num, den = acc_ref[:, :E], acc_ref[:, E:]
        outm = (num + eon * von_ref[0]) / (den + eon)
        out_ref[0] = jnp.dot(outm, wo_ref[...],
                             preferred_element_type=jnp.float32)


def kernel(xc_off_grid, xc_on_grid, zc_off_grid, zc_on_grid, ignore_on_grid,
           latents, fake_embedding, Wq, Wk, Wv, Wo):
    Bv = xc_on_grid.shape[0]
    grid_shape = xc_on_grid.shape[1:-1]
    zc_on = zc_on_grid.reshape(Bv, S, E)
    on_tok = jnp.where(jnp.asarray(ignore_on_grid),
                       jnp.broadcast_to(fake_embedding, (Bv, S, E)), zc_on)

    qm, eon, von = pl.pallas_call(
        _prep_kernel,
        grid=(Bv,),
        in_specs=[
            pl.BlockSpec((S, E), lambda b: (0, 0)),
            pl.BlockSpec((1, S, E), lambda b: (b, 0, 0)),
            pl.BlockSpec((E, E), lambda b: (0, 0)),
            pl.BlockSpec((E, E), lambda b: (0, 0)),
            pl.BlockSpec((E, E), lambda b: (0, 0)),
        ],
        out_specs=[
            pl.BlockSpec((S, E), lambda b: (0, 0)),
            pl.BlockSpec((1, S, E), lambda b: (b, 0, 0)),
            pl.BlockSpec((1, S, E), lambda b: (b, 0, 0)),
        ],
        out_shape=[
            jax.ShapeDtypeStruct((S, E), jnp.float32),
            jax.ShapeDtypeStruct((Bv, S, E), jnp.float32),
            jax.ShapeDtypeStruct((Bv, S, E), jnp.float32),
        ],
    )(latents, on_tok, Wq, Wk, Wv)

    x_flat = xc_off_grid[..., 0].reshape(Bv * U)
    y_flat = xc_off_grid[..., 1].reshape(Bv * U)
    qg_flat, idx_flat = _sc_gather(x_flat, y_flat, qm)

    qg = qg_flat.reshape(Bv, U, E)
    idx4 = idx_flat.reshape(Bv, NU, 1, BU)

    out = pl.pallas_call(
        _main_kernel,
        grid=(Bv, NU),
        in_specs=[
            pl.BlockSpec((1, BU, E), lambda b, u: (b, u, 0)),
            pl.BlockSpec((1, 1, 1, BU), lambda b, u: (b, u, 0, 0)),
            pl.BlockSpec((1, BU, E), lambda b, u: (b, u, 0)),
            pl.BlockSpec((1, S, E), lambda b, u: (b, 0, 0)),
            pl.BlockSpec((1, S, E), lambda b, u: (b, 0, 0)),
            pl.BlockSpec((E, E), lambda b, u: (0, 0)),
            pl.BlockSpec((E, E), lambda b, u: (0, 0)),
            pl.BlockSpec((E, E), lambda b, u: (0, 0)),
        ],
        out_specs=pl.BlockSpec((1, S, E), lambda b, u: (b, 0, 0)),
        out_shape=jax.ShapeDtypeStruct((Bv, S, E), jnp.float32),
        scratch_shapes=[
            pltpu.VMEM((S, 2 * E), jnp.float32),
            pltpu.VMEM((E, 2 * E), jnp.float32),
        ],
        compiler_params=pltpu.CompilerParams(
            dimension_semantics=("parallel", "arbitrary")),
    )(qg, idx4, zc_off_grid, eon, von, Wk, Wv, Wo)

    return out.reshape((Bv,) + tuple(grid_shape) + (E,))


# speedup vs baseline: 1.0821x; 1.0099x over previous
import functools

import jax
from jax import lax
import jax.numpy as jnp
import numpy as np
from jax.experimental import pallas as pl
from jax.experimental.pallas import tpu as pltpu
from jax.experimental.pallas import tpu_sc as plsc

B, U, GH, GW, E, DX, H = 4, 8192, 32, 32, 128, 2, 8
S = GH * GW
DH = E // H
BU = 4096
NU = U // BU
INV_SQRT_DH = 1.0 / np.sqrt(DH)

_SC_INFO = plsc.get_sparse_core_info()
NC, NS, NL = _SC_INFO.num_cores, _SC_INFO.num_subcores, _SC_INFO.num_lanes
NW = NC * NS
TPW = (B * U) // NW
CHUNK = 128
NCH = TPW // CHUNK


def _head_mask():
    r = jax.lax.broadcasted_iota(jnp.int32, (E, E), 0) // DH
    c = jax.lax.broadcasted_iota(jnp.int32, (E, E), 1) // DH
    return (r == c).astype(jnp.float32)


def _prep_kernel(lat_ref, on_ref, wq_ref, wk_ref, wv_ref,
                 qm_ref, eon_ref, von_ref):
    b = pl.program_id(0)
    mhead = _head_mask()
    qm = jnp.dot(lat_ref[...], wq_ref[...], preferred_element_type=jnp.float32)

    @pl.when(b == 0)
    def _():
        qm_ref[...] = qm

    on = on_ref[0]
    kon = jnp.dot(on, wk_ref[...], preferred_element_type=jnp.float32)
    von_ref[0] = jnp.dot(on, wv_ref[...], preferred_element_type=jnp.float32)
    son = jnp.dot(qm * kon, mhead,
                  preferred_element_type=jnp.float32) * INV_SQRT_DH
    eon_ref[0] = jnp.exp(son)


_sc_mesh = plsc.VectorSubcoreMesh(core_axis_name="c", subcore_axis_name="s")


@functools.partial(
    pl.kernel, mesh=_sc_mesh,
    out_type=[jax.ShapeDtypeStruct((B * U, E), jnp.float32),
              jax.ShapeDtypeStruct((B * U,), jnp.int32)],
    scratch_types=[
        pltpu.VMEM((TPW,), jnp.float32),
        pltpu.VMEM((TPW,), jnp.float32),
        pltpu.VMEM((TPW,), jnp.int32),
        pltpu.VMEM((CHUNK, E), jnp.float32),
        pltpu.VMEM((CHUNK, E), jnp.float32),
        pltpu.SemaphoreType.DMA,
        pltpu.SemaphoreType.DMA,
    ],
)
def _sc_gather(x_hbm, y_hbm, qm_hbm, qg_hbm, idx_hbm,
               x_v, y_v, idx_v, rows0, rows1, sem0, sem1):
    wid = lax.axis_index("s") * NC + lax.axis_index("c")
    base = wid * TPW
    pltpu.sync_copy(x_hbm.at[pl.ds(base, TPW)], x_v)
    pltpu.sync_copy(y_hbm.at[pl.ds(base, TPW)], y_v)
    for i in range(TPW // NL):
        sl = pl.ds(i * NL, NL)
        tx = jnp.minimum(jnp.maximum(x_v[sl] * (GH - 1) + 0.5, 0.0),
                         GH - 1 + 0.49)
        ty = jnp.minimum(jnp.maximum(y_v[sl] * (GW - 1) + 0.5, 0.0),
                         GW - 1 + 0.49)
        gi = tx.astype(jnp.int32)
        gj = ty.astype(jnp.int32)
        idx_v[sl] = gi * GW + gj
    pltpu.sync_copy(idx_v, idx_hbm.at[pl.ds(base, TPW)])
    rows = (rows0, rows1)
    sems = (sem0, sem1)
    handles = [None, None]
    for ci in range(NCH):
        bsel = ci % 2
        handles[bsel] = pltpu.async_copy(
            qm_hbm.at[idx_v.at[pl.ds(ci * CHUNK, CHUNK)]], rows[bsel],
            sems[bsel])
        if ci >= 1:
            pb = (ci - 1) % 2
            handles[pb].wait()
            pltpu.sync_copy(rows[pb],
                            qg_hbm.at[pl.ds(base + (ci - 1) * CHUNK, CHUNK)])
    last = (NCH - 1) % 2
    handles[last].wait()
    pltpu.sync_copy(rows[last],
                    qg_hbm.at[pl.ds(base + (NCH - 1) * CHUNK, CHUNK)])


def _main_kernel(qg_ref, idx_ref, z_ref, eon_ref, von_ref, wk_ref, wv_ref,
                 wo_ref, out_ref, acc_ref, wkv_ref):
    u = pl.program_id(1)
    mhead = _head_mask()

    @pl.when(u == 0)
    def _init():
        wkv_ref[...] = jnp.concatenate([wk_ref[...], wv_ref[...]], axis=1)
        acc_ref[...] = jnp.zeros_like(acc_ref)

    z = z_ref[0]
    qg = qg_ref[0]
    idx_r = idx_ref[0, 0]
    onehot_t = (idx_r == jax.lax.broadcasted_iota(jnp.int32, (S, BU), 0)
                ).astype(jnp.float32)

    kv = jnp.dot(z, wkv_ref[...], preferred_element_type=jnp.float32)
    k, v = kv[:, :E], kv[:, E:]
    scores = jnp.dot(qg * k, mhead,
                     preferred_element_type=jnp.float32) * INV_SQRT_DH
    w = jnp.exp(scores)

    payload = jnp.concatenate([v * w, w], axis=1)
    acc_ref[...] += jnp.dot(onehot_t, payload,
                            preferred_element_type=jnp.float32)

    @pl.when(u == NU - 1)
    def _finalize():
        eon = eon_ref[0]
        num, den = acc_ref[:, :E], acc_ref[:, E:]
        outm = (num + eon * von_ref[0]) / (den + eon)
        out_ref[0] = jnp.dot(outm, wo_ref[...],
                             preferred_element_type=jnp.float32)


def kernel(xc_off_grid, xc_on_grid, zc_off_grid, zc_on_grid, ignore_on_grid,
           latents, fake_embedding, Wq, Wk, Wv, Wo):
    Bv = xc_on_grid.shape[0]
    grid_shape = xc_on_grid.shape[1:-1]
    zc_on = zc_on_grid.reshape(Bv, S, E)
    on_tok = jnp.where(jnp.asarray(ignore_on_grid),
                       jnp.broadcast_to(fake_embedding, (Bv, S, E)), zc_on)

    qm, eon, von = pl.pallas_call(
        _prep_kernel,
        grid=(Bv,),
        in_specs=[
            pl.BlockSpec((S, E), lambda b: (0, 0)),
            pl.BlockSpec((1, S, E), lambda b: (b, 0, 0)),
            pl.BlockSpec((E, E), lambda b: (0, 0)),
            pl.BlockSpec((E, E), lambda b: (0, 0)),
            pl.BlockSpec((E, E), lambda b: (0, 0)),
        ],
        out_specs=[
            pl.BlockSpec((S, E), lambda b: (0, 0)),
            pl.BlockSpec((1, S, E), lambda b: (b, 0, 0)),
            pl.BlockSpec((1, S, E), lambda b: (b, 0, 0)),
        ],
        out_shape=[
            jax.ShapeDtypeStruct((S, E), jnp.float32),
            jax.ShapeDtypeStruct((Bv, S, E), jnp.float32),
            jax.ShapeDtypeStruct((Bv, S, E), jnp.float32),
        ],
    )(latents, on_tok, Wq, Wk, Wv)

    x_flat = xc_off_grid[..., 0].reshape(Bv * U)
    y_flat = xc_off_grid[..., 1].reshape(Bv * U)
    qg_flat, idx_flat = _sc_gather(x_flat, y_flat, qm)

    qg = qg_flat.reshape(Bv, U, E)
    idx4 = idx_flat.reshape(Bv, NU, 1, BU)

    out = pl.pallas_call(
        _main_kernel,
        grid=(Bv, NU),
        in_specs=[
            pl.BlockSpec((1, BU, E), lambda b, u: (b, u, 0)),
            pl.BlockSpec((1, 1, 1, BU), lambda b, u: (b, u, 0, 0)),
            pl.BlockSpec((1, BU, E), lambda b, u: (b, u, 0)),
            pl.BlockSpec((1, S, E), lambda b, u: (b, 0, 0)),
            pl.BlockSpec((1, S, E), lambda b, u: (b, 0, 0)),
            pl.BlockSpec((E, E), lambda b, u: (0, 0)),
            pl.BlockSpec((E, E), lambda b, u: (0, 0)),
            pl.BlockSpec((E, E), lambda b, u: (0, 0)),
        ],
        out_specs=pl.BlockSpec((1, S, E), lambda b, u: (b, 0, 0)),
        out_shape=jax.ShapeDtypeStruct((Bv, S, E), jnp.float32),
        scratch_shapes=[
            pltpu.VMEM((S, 2 * E), jnp.float32),
            pltpu.VMEM((E, 2 * E), jnp.float32),
        ],
        compiler_params=pltpu.CompilerParams(
            dimension_semantics=("parallel", "arbitrary")),
    )(qg, idx4, zc_off_grid, eon, von, Wk, Wv, Wo)

    return out.reshape((Bv,) + tuple(grid_shape) + (E,))


# speedup vs baseline: 1.0913x; 1.0085x over previous
import functools

import jax
from jax import lax
import jax.numpy as jnp
import numpy as np
from jax.experimental import pallas as pl
from jax.experimental.pallas import tpu as pltpu
from jax.experimental.pallas import tpu_sc as plsc

B, U, GH, GW, E, DX, H = 4, 8192, 32, 32, 128, 2, 8
S = GH * GW
DH = E // H
BU = 8192
NU = U // BU
INV_SQRT_DH = 1.0 / np.sqrt(DH)

_SC_INFO = plsc.get_sparse_core_info()
NC, NS, NL = _SC_INFO.num_cores, _SC_INFO.num_subcores, _SC_INFO.num_lanes
NW = NC * NS
TPW = (B * U) // NW
CHUNK = 128
NCH = TPW // CHUNK


def _head_mask():
    r = jax.lax.broadcasted_iota(jnp.int32, (E, E), 0) // DH
    c = jax.lax.broadcasted_iota(jnp.int32, (E, E), 1) // DH
    return (r == c).astype(jnp.float32)


def _prep_kernel(lat_ref, on_ref, wq_ref, wk_ref, wv_ref,
                 qm_ref, eon_ref, von_ref):
    b = pl.program_id(0)
    mhead = _head_mask()
    qm = jnp.dot(lat_ref[...], wq_ref[...], preferred_element_type=jnp.float32)

    @pl.when(b == 0)
    def _():
        qm_ref[...] = qm

    on = on_ref[0]
    kon = jnp.dot(on, wk_ref[...], preferred_element_type=jnp.float32)
    von_ref[0] = jnp.dot(on, wv_ref[...], preferred_element_type=jnp.float32)
    son = jnp.dot(qm * kon, mhead,
                  preferred_element_type=jnp.float32) * INV_SQRT_DH
    eon_ref[0] = jnp.exp(son)


_sc_mesh = plsc.VectorSubcoreMesh(core_axis_name="c", subcore_axis_name="s")


@functools.partial(
    pl.kernel, mesh=_sc_mesh,
    out_type=[jax.ShapeDtypeStruct((B * U, E), jnp.float32),
              jax.ShapeDtypeStruct((B * U,), jnp.int32)],
    scratch_types=[
        pltpu.VMEM((TPW,), jnp.float32),
        pltpu.VMEM((TPW,), jnp.float32),
        pltpu.VMEM((TPW,), jnp.int32),
        pltpu.VMEM((CHUNK, E), jnp.float32),
        pltpu.VMEM((CHUNK, E), jnp.float32),
        pltpu.SemaphoreType.DMA,
        pltpu.SemaphoreType.DMA,
    ],
)
def _sc_gather(x_hbm, y_hbm, qm_hbm, qg_hbm, idx_hbm,
               x_v, y_v, idx_v, rows0, rows1, sem0, sem1):
    wid = lax.axis_index("s") * NC + lax.axis_index("c")
    base = wid * TPW
    pltpu.sync_copy(x_hbm.at[pl.ds(base, TPW)], x_v)
    pltpu.sync_copy(y_hbm.at[pl.ds(base, TPW)], y_v)
    for i in range(TPW // NL):
        sl = pl.ds(i * NL, NL)
        tx = jnp.minimum(jnp.maximum(x_v[sl] * (GH - 1) + 0.5, 0.0),
                         GH - 1 + 0.49)
        ty = jnp.minimum(jnp.maximum(y_v[sl] * (GW - 1) + 0.5, 0.0),
                         GW - 1 + 0.49)
        gi = tx.astype(jnp.int32)
        gj = ty.astype(jnp.int32)
        idx_v[sl] = gi * GW + gj
    pltpu.sync_copy(idx_v, idx_hbm.at[pl.ds(base, TPW)])
    rows = (rows0, rows1)
    sems = (sem0, sem1)
    handles = [None, None]
    for ci in range(NCH):
        bsel = ci % 2
        handles[bsel] = pltpu.async_copy(
            qm_hbm.at[idx_v.at[pl.ds(ci * CHUNK, CHUNK)]], rows[bsel],
            sems[bsel])
        if ci >= 1:
            pb = (ci - 1) % 2
            handles[pb].wait()
            pltpu.sync_copy(rows[pb],
                            qg_hbm.at[pl.ds(base + (ci - 1) * CHUNK, CHUNK)])
    last = (NCH - 1) % 2
    handles[last].wait()
    pltpu.sync_copy(rows[last],
                    qg_hbm.at[pl.ds(base + (NCH - 1) * CHUNK, CHUNK)])


def _main_kernel(qg_ref, idx_ref, z_ref, eon_ref, von_ref, wk_ref, wv_ref,
                 wo_ref, out_ref, acc_ref, wkv_ref):
    u = pl.program_id(1)
    mhead = _head_mask()

    @pl.when(u == 0)
    def _init():
        wkv_ref[...] = jnp.concatenate([wk_ref[...], wv_ref[...]], axis=1)
        acc_ref[...] = jnp.zeros_like(acc_ref)

    z = z_ref[0]
    qg = qg_ref[0]
    idx_r = idx_ref[0, 0]
    onehot_t = (idx_r == jax.lax.broadcasted_iota(jnp.int32, (S, BU), 0)
                ).astype(jnp.float32)

    kv = jnp.dot(z, wkv_ref[...], preferred_element_type=jnp.float32)
    k, v = kv[:, :E], kv[:, E:]
    scores = jnp.dot(qg * k, mhead,
                     preferred_element_type=jnp.float32) * INV_SQRT_DH
    w = jnp.exp(scores)

    payload = jnp.concatenate([v * w, w], axis=1)
    acc_ref[...] += jnp.dot(onehot_t, payload,
                            preferred_element_type=jnp.float32)

    @pl.when(u == NU - 1)
    def _finalize():
        eon = eon_ref[0]
        num, den = acc_ref[:, :E], acc_ref[:, E:]
        outm = (num + eon * von_ref[0]) / (den + eon)
        out_ref[0] = jnp.dot(outm, wo_ref[...],
                             preferred_element_type=jnp.float32)


def kernel(xc_off_grid, xc_on_grid, zc_off_grid, zc_on_grid, ignore_on_grid,
           latents, fake_embedding, Wq, Wk, Wv, Wo):
    Bv = xc_on_grid.shape[0]
    grid_shape = xc_on_grid.shape[1:-1]
    zc_on = zc_on_grid.reshape(Bv, S, E)
    on_tok = jnp.where(jnp.asarray(ignore_on_grid),
                       jnp.broadcast_to(fake_embedding, (Bv, S, E)), zc_on)

    qm, eon, von = pl.pallas_call(
        _prep_kernel,
        grid=(Bv,),
        in_specs=[
            pl.BlockSpec((S, E), lambda b: (0, 0)),
            pl.BlockSpec((1, S, E), lambda b: (b, 0, 0)),
            pl.BlockSpec((E, E), lambda b: (0, 0)),
            pl.BlockSpec((E, E), lambda b: (0, 0)),
            pl.BlockSpec((E, E), lambda b: (0, 0)),
        ],
        out_specs=[
            pl.BlockSpec((S, E), lambda b: (0, 0)),
            pl.BlockSpec((1, S, E), lambda b: (b, 0, 0)),
            pl.BlockSpec((1, S, E), lambda b: (b, 0, 0)),
        ],
        out_shape=[
            jax.ShapeDtypeStruct((S, E), jnp.float32),
            jax.ShapeDtypeStruct((Bv, S, E), jnp.float32),
            jax.ShapeDtypeStruct((Bv, S, E), jnp.float32),
        ],
    )(latents, on_tok, Wq, Wk, Wv)

    x_flat = xc_off_grid[..., 0].reshape(Bv * U)
    y_flat = xc_off_grid[..., 1].reshape(Bv * U)
    qg_flat, idx_flat = _sc_gather(x_flat, y_flat, qm)

    qg = qg_flat.reshape(Bv, U, E)
    idx4 = idx_flat.reshape(Bv, NU, 1, BU)

    out = pl.pallas_call(
        _main_kernel,
        grid=(Bv, NU),
        in_specs=[
            pl.BlockSpec((1, BU, E), lambda b, u: (b, u, 0)),
            pl.BlockSpec((1, 1, 1, BU), lambda b, u: (b, u, 0, 0)),
            pl.BlockSpec((1, BU, E), lambda b, u: (b, u, 0)),
            pl.BlockSpec((1, S, E), lambda b, u: (b, 0, 0)),
            pl.BlockSpec((1, S, E), lambda b, u: (b, 0, 0)),
            pl.BlockSpec((E, E), lambda b, u: (0, 0)),
            pl.BlockSpec((E, E), lambda b, u: (0, 0)),
            pl.BlockSpec((E, E), lambda b, u: (0, 0)),
        ],
        out_specs=pl.BlockSpec((1, S, E), lambda b, u: (b, 0, 0)),
        out_shape=jax.ShapeDtypeStruct((Bv, S, E), jnp.float32),
        scratch_shapes=[
            pltpu.VMEM((S, 2 * E), jnp.float32),
            pltpu.VMEM((E, 2 * E), jnp.float32),
        ],
        compiler_params=pltpu.CompilerParams(
            dimension_semantics=("parallel", "arbitrary")),
    )(qg, idx4, zc_off_grid, eon, von, Wk, Wv, Wo)

    return out.reshape((Bv,) + tuple(grid_shape) + (E,))


# speedup vs baseline: 1.1046x; 1.0122x over previous
import functools

import jax
from jax import lax
import jax.numpy as jnp
import numpy as np
from jax.experimental import pallas as pl
from jax.experimental.pallas import tpu as pltpu
from jax.experimental.pallas import tpu_sc as plsc

B, U, GH, GW, E, DX, H = 4, 8192, 32, 32, 128, 2, 8
S = GH * GW
DH = E // H
BU = 8192
NU = U // BU
INV_SQRT_DH = 1.0 / np.sqrt(DH)

_SC_INFO = plsc.get_sparse_core_info()
NC, NS, NL = _SC_INFO.num_cores, _SC_INFO.num_subcores, _SC_INFO.num_lanes
NW = NC * NS
TPW = (B * U) // NW
CHUNK = 128
NCH = TPW // CHUNK


def _head_mask():
    r = jax.lax.broadcasted_iota(jnp.int32, (E, E), 0) // DH
    c = jax.lax.broadcasted_iota(jnp.int32, (E, E), 1) // DH
    return (r == c).astype(jnp.float32)


def _prep_kernel(lat_ref, on_ref, wq_ref, wk_ref, wv_ref,
                 qm_ref, eon_ref, von_ref):
    b = pl.program_id(0)
    mhead = _head_mask()
    qm = jnp.dot(lat_ref[...], wq_ref[...], preferred_element_type=jnp.float32)

    @pl.when(b == 0)
    def _():
        qm_ref[...] = qm

    on = on_ref[0]
    kon = jnp.dot(on, wk_ref[...], preferred_element_type=jnp.float32)
    von_ref[0] = jnp.dot(on, wv_ref[...], preferred_element_type=jnp.float32)
    son = jnp.dot(qm * kon, mhead,
                  preferred_element_type=jnp.float32) * INV_SQRT_DH
    eon_ref[0] = jnp.exp(son)


_sc_mesh = plsc.VectorSubcoreMesh(core_axis_name="c", subcore_axis_name="s")


@functools.partial(
    pl.kernel, mesh=_sc_mesh,
    out_type=[jax.ShapeDtypeStruct((B * U, E), jnp.float32),
              jax.ShapeDtypeStruct((B * U,), jnp.int32)],
    scratch_types=[
        pltpu.VMEM((TPW,), jnp.float32),
        pltpu.VMEM((TPW,), jnp.float32),
        pltpu.VMEM((TPW,), jnp.int32),
        pltpu.VMEM((CHUNK, E), jnp.float32),
        pltpu.VMEM((CHUNK, E), jnp.float32),
        pltpu.VMEM((CHUNK, E), jnp.float32),
        pltpu.VMEM((CHUNK, E), jnp.float32),
        pltpu.SemaphoreType.DMA,
        pltpu.SemaphoreType.DMA,
        pltpu.SemaphoreType.DMA,
        pltpu.SemaphoreType.DMA,
    ],
)
def _sc_gather(x_hbm, y_hbm, qm_hbm, qg_hbm, idx_hbm,
               x_v, y_v, idx_v, rows0, rows1, rows2, rows3,
               sem0, sem1, sem2, sem3):
    wid = lax.axis_index("s") * NC + lax.axis_index("c")
    base = wid * TPW
    pltpu.sync_copy(x_hbm.at[pl.ds(base, TPW)], x_v)
    pltpu.sync_copy(y_hbm.at[pl.ds(base, TPW)], y_v)
    for i in range(TPW // NL):
        sl = pl.ds(i * NL, NL)
        tx = jnp.minimum(jnp.maximum(x_v[sl] * (GH - 1) + 0.5, 0.0),
                         GH - 1 + 0.49)
        ty = jnp.minimum(jnp.maximum(y_v[sl] * (GW - 1) + 0.5, 0.0),
                         GW - 1 + 0.49)
        gi = tx.astype(jnp.int32)
        gj = ty.astype(jnp.int32)
        idx_v[sl] = gi * GW + gj
    pltpu.sync_copy(idx_v, idx_hbm.at[pl.ds(base, TPW)])
    rows = (rows0, rows1, rows2, rows3)
    sems = (sem0, sem1, sem2, sem3)
    depth = 4
    handles = [None] * depth
    for ci in range(NCH):
        bsel = ci % depth
        handles[bsel] = pltpu.async_copy(
            qm_hbm.at[idx_v.at[pl.ds(ci * CHUNK, CHUNK)]], rows[bsel],
            sems[bsel])
        if ci >= depth - 1:
            pc = ci - (depth - 1)
            pb = pc % depth
            handles[pb].wait()
            pltpu.sync_copy(rows[pb],
                            qg_hbm.at[pl.ds(base + pc * CHUNK, CHUNK)])
    for pc in range(max(0, NCH - (depth - 1)), NCH):
        pb = pc % depth
        handles[pb].wait()
        pltpu.sync_copy(rows[pb],
                        qg_hbm.at[pl.ds(base + pc * CHUNK, CHUNK)])


def _main_kernel(qg_ref, idx_ref, z_ref, eon_ref, von_ref, wk_ref, wv_ref,
                 wo_ref, out_ref, acc_ref, wkv_ref):
    u = pl.program_id(1)
    mhead = _head_mask()

    @pl.when(u == 0)
    def _init():
        wkv_ref[...] = jnp.concatenate([wk_ref[...], wv_ref[...]], axis=1)
        acc_ref[...] = jnp.zeros_like(acc_ref)

    z = z_ref[0]
    qg = qg_ref[0]
    idx_r = idx_ref[0, 0]
    onehot_t = (idx_r == jax.lax.broadcasted_iota(jnp.int32, (S, BU), 0)
                ).astype(jnp.float32)

    kv = jnp.dot(z, wkv_ref[...], preferred_element_type=jnp.float32)
    k, v = kv[:, :E], kv[:, E:]
    scores = jnp.dot(qg * k, mhead,
                     preferred_element_type=jnp.float32) * INV_SQRT_DH
    w = jnp.exp(scores)

    payload = jnp.concatenate([v * w, w], axis=1)
    acc_ref[...] += jnp.dot(onehot_t, payload,
                            preferred_element_type=jnp.float32)

    @pl.when(u == NU - 1)
    def _finalize():
        eon = eon_ref[0]
        num, den = acc_ref[:, :E], acc_ref[:, E:]
        outm = (num + eon * von_ref[0]) / (den + eon)
        out_ref[0] = jnp.dot(outm, wo_ref[...],
                             preferred_element_type=jnp.float32)


def kernel(xc_off_grid, xc_on_grid, zc_off_grid, zc_on_grid, ignore_on_grid,
           latents, fake_embedding, Wq, Wk, Wv, Wo):
    Bv = xc_on_grid.shape[0]
    grid_shape = xc_on_grid.shape[1:-1]
    zc_on = zc_on_grid.reshape(Bv, S, E)
    on_tok = jnp.where(jnp.asarray(ignore_on_grid),
                       jnp.broadcast_to(fake_embedding, (Bv, S, E)), zc_on)

    qm, eon, von = pl.pallas_call(
        _prep_kernel,
        grid=(Bv,),
        in_specs=[
            pl.BlockSpec((S, E), lambda b: (0, 0)),
            pl.BlockSpec((1, S, E), lambda b: (b, 0, 0)),
            pl.BlockSpec((E, E), lambda b: (0, 0)),
            pl.BlockSpec((E, E), lambda b: (0, 0)),
            pl.BlockSpec((E, E), lambda b: (0, 0)),
        ],
        out_specs=[
            pl.BlockSpec((S, E), lambda b: (0, 0)),
            pl.BlockSpec((1, S, E), lambda b: (b, 0, 0)),
            pl.BlockSpec((1, S, E), lambda b: (b, 0, 0)),
        ],
        out_shape=[
            jax.ShapeDtypeStruct((S, E), jnp.float32),
            jax.ShapeDtypeStruct((Bv, S, E), jnp.float32),
            jax.ShapeDtypeStruct((Bv, S, E), jnp.float32),
        ],
    )(latents, on_tok, Wq, Wk, Wv)

    x_flat = xc_off_grid[..., 0].reshape(Bv * U)
    y_flat = xc_off_grid[..., 1].reshape(Bv * U)
    qg_flat, idx_flat = _sc_gather(x_flat, y_flat, qm)

    qg = qg_flat.reshape(Bv, U, E)
    idx4 = idx_flat.reshape(Bv, NU, 1, BU)

    out = pl.pallas_call(
        _main_kernel,
        grid=(Bv, NU),
        in_specs=[
            pl.BlockSpec((1, BU, E), lambda b, u: (b, u, 0)),
            pl.BlockSpec((1, 1, 1, BU), lambda b, u: (b, u, 0, 0)),
            pl.BlockSpec((1, BU, E), lambda b, u: (b, u, 0)),
            pl.BlockSpec((1, S, E), lambda b, u: (b, 0, 0)),
            pl.BlockSpec((1, S, E), lambda b, u: (b, 0, 0)),
            pl.BlockSpec((E, E), lambda b, u: (0, 0)),
            pl.BlockSpec((E, E), lambda b, u: (0, 0)),
            pl.BlockSpec((E, E), lambda b, u: (0, 0)),
        ],
        out_specs=pl.BlockSpec((1, S, E), lambda b, u: (b, 0, 0)),
        out_shape=jax.ShapeDtypeStruct((Bv, S, E), jnp.float32),
        scratch_shapes=[
            pltpu.VMEM((S, 2 * E), jnp.float32),
            pltpu.VMEM((E, 2 * E), jnp.float32),
        ],
        compiler_params=pltpu.CompilerParams(
            dimension_semantics=("parallel", "arbitrary")),
    )(qg, idx4, zc_off_grid, eon, von, Wk, Wv, Wo)

    return out.reshape((Bv,) + tuple(grid_shape) + (E,))
